# Initial kernel scaffold; baseline (speedup 1.0000x reference)
import jax
import jax.numpy as jnp
from jax.experimental import pallas as pl


def kernel(x, edge_index, W1, b1, W2, b2, W3, b3, W4, b4, W5, b5):
    raise NotImplementedError("write your pallas kernel here")


# speedup vs baseline: 6.1505x; 6.1505x over previous
import functools

import jax
import jax.numpy as jnp
from jax import lax
from jax.experimental import pallas as pl
from jax.experimental.pallas import tpu as pltpu
from jax.experimental.pallas import tpu_sc as plsc

N = 10000
E = 320000
F_IN = 128
DIM = 64
C = 16

NC = 2
NS = 16
NW = NC * NS
K = 128
NCHUNK = E // K
SLOTS = -(-NCHUNK // NW)
NPAD = SLOTS * K
ZCH = NPAD // K
ZSLOTS = -(-ZCH // NS)


def _make_agg(F):
    mesh = plsc.VectorSubcoreMesh(core_axis_name="c", subcore_axis_name="s")

    @functools.partial(
        pl.kernel,
        out_type=jax.ShapeDtypeStruct((NC, NPAD, F), jnp.float32),
        mesh=mesh,
        scratch_types=[
            pltpu.VMEM((K,), jnp.int32),
            pltpu.VMEM((K,), jnp.int32),
            pltpu.VMEM((K, F), jnp.float32),
            pltpu.VMEM((K, F), jnp.float32),
            pltpu.VMEM_SHARED((NPAD, F), jnp.float32),
            pltpu.SemaphoreType.DMA,
        ],
        compiler_params=pltpu.CompilerParams(use_tc_tiling_on_sc=False),
    )
    def agg_kernel(x_hbm, edge_hbm, out_hbm, src_v, dst_v, rows_v, zeros_v,
                   acc_sh, sem):
        cid = lax.axis_index("c")
        sid = lax.axis_index("s")
        wid = sid * NC + cid

        zvec = jnp.zeros((16,), jnp.float32)

        def zero_row(i, _):
            for j in range(F // 16):
                zeros_v[i, pl.ds(16 * j, 16)] = zvec
            return 0

        lax.fori_loop(0, K, zero_row, 0)

        def zero_acc(c, _):
            cc = sid + NS * c

            @pl.when(cc < ZCH)
            def _():
                pltpu.sync_copy(zeros_v, acc_sh.at[pl.ds(cc * K, K)])

            return 0

        lax.fori_loop(0, ZSLOTS, zero_acc, 0)
        plsc.subcore_barrier()

        def do_chunk(c, _):
            cc = wid + NW * c

            @pl.when(cc < NCHUNK)
            def _():
                pltpu.sync_copy(edge_hbm.at[0, pl.ds(cc * K, K)], src_v)
                pltpu.sync_copy(edge_hbm.at[1, pl.ds(cc * K, K)], dst_v)
                pltpu.async_copy(x_hbm.at[src_v], rows_v, sem).wait()
                pltpu.sync_copy(rows_v, acc_sh.at[dst_v], add=True)

            return 0

        lax.fori_loop(0, SLOTS, do_chunk, 0)
        plsc.subcore_barrier()

        def copy_out(c, _):
            cc = sid + NS * c

            @pl.when(cc < ZCH)
            def _():
                pltpu.sync_copy(acc_sh.at[pl.ds(cc * K, K)],
                                out_hbm.at[cid, pl.ds(cc * K, K)])

            return 0

        lax.fori_loop(0, ZSLOTS, copy_out, 0)

    return agg_kernel


def _make_mm(F_in, F_out):
    def mm_body(x_ref, a_ref, w_ref, b_ref, o_ref):
        h = x_ref[...] + a_ref[0, :N, :] + a_ref[1, :N, :]
        o_ref[...] = (
            jnp.dot(h, w_ref[...], preferred_element_type=jnp.float32)
            + b_ref[...]
        )

    return pl.pallas_call(
        mm_body,
        out_shape=jax.ShapeDtypeStruct((N, F_out), jnp.float32),
        in_specs=[
            pl.BlockSpec(memory_space=pltpu.VMEM),
            pl.BlockSpec(memory_space=pltpu.VMEM),
            pl.BlockSpec(memory_space=pltpu.VMEM),
            pl.BlockSpec(memory_space=pltpu.VMEM),
        ],
        out_specs=pl.BlockSpec(memory_space=pltpu.VMEM),
    )


_agg128 = _make_agg(F_IN)
_agg64 = _make_agg(DIM)
_mm1 = _make_mm(F_IN, DIM)
_mm_mid = _make_mm(DIM, DIM)
_mm5 = _make_mm(DIM, C)


def kernel(x, edge_index, W1, b1, W2, b2, W3, b3, W4, b4, W5, b5):
    edge_index = edge_index.astype(jnp.int32)

    def layer(agg_fn, mm_fn, h, W, b):
        parts = agg_fn(h, edge_index)
        return mm_fn(h, parts, W, b.reshape(1, -1))

    h = layer(_agg128, _mm1, x, W1, b1)
    h = layer(_agg64, _mm_mid, h, W2, b2)
    h = layer(_agg64, _mm_mid, h, W3, b3)
    h = layer(_agg64, _mm_mid, h, W4, b4)
    h = layer(_agg64, _mm5, h, W5, b5)
    return h
